# Initial kernel scaffold; baseline (speedup 1.0000x reference)
import jax
import jax.numpy as jnp
from jax.experimental import pallas as pl


def kernel(x, edge_index, batch, Wp, bp, W0, b0, W1, b1, W2, b2, Wo, bo):
    raise NotImplementedError("write your pallas kernel here")


# speedup vs baseline: 9.1461x; 9.1461x over previous
import functools

import jax
import jax.numpy as jnp
from jax import lax
from jax.experimental import pallas as pl
from jax.experimental.pallas import tpu as pltpu
from jax.experimental.pallas import tpu_sc as plsc

N = 10000
E = 320000
D_IN = 128
H = 256
HH = H // 2
NC = 2
NS = 16
L = 16
CH = 128
G_TOTAL = E // CH
K_MAX = (G_TOTAL + NS - 1) // NS

ROWS_BIG = 640
ROWS_LAST = N - 15 * ROWS_BIG

NP = 10240
ROWS_P = NP // NS

_mesh = plsc.VectorSubcoreMesh(core_axis_name="c", subcore_axis_name="s",
                               num_cores=NC, num_subcores=NS)


def _fill(ref, n, value, dtype=jnp.float32):
    v = jnp.full((L,), value, dtype)
    for i in range(n // L):
        ref[pl.ds(i * L, L)] = v


def _rows_init(copy_fn, s):
    @pl.when(s < 15)
    def _():
        copy_fn(s * ROWS_BIG, ROWS_BIG)

    @pl.when(s == 15)
    def _():
        copy_fn(15 * ROWS_BIG, ROWS_LAST)


@functools.partial(
    pl.kernel,
    out_type=jax.ShapeDtypeStruct((NC, NP), jnp.float32),
    mesh=_mesh,
    scratch_types=[
        pltpu.VMEM_SHARED((NP,), jnp.float32),
        pltpu.VMEM((ROWS_P,), jnp.float32),
        pltpu.VMEM((CH,), jnp.int32),
        pltpu.VMEM((CH,), jnp.float32),
    ],
)
def _deg_kernel(dst_hbm, deg_hbm, acc_sh, init_v, idx_v, ones_v):
    c = lax.axis_index("c")
    s = lax.axis_index("s")

    fill_val = jnp.where(c == 0, 1.0, 0.0)
    v = jnp.full((L,), 1.0, jnp.float32) * fill_val
    for i in range(ROWS_P // L):
        init_v[pl.ds(i * L, L)] = v
    _fill(ones_v, CH, 1.0)

    pltpu.sync_copy(init_v, acc_sh.at[pl.ds(s * ROWS_P, ROWS_P)])
    plsc.subcore_barrier()

    g0 = c * (G_TOTAL // NC)
    g1 = (c + 1) * (G_TOTAL // NC)

    def body(k, _):
        g = g0 + s + k * NS

        @pl.when(g < g1)
        def _():
            pltpu.sync_copy(dst_hbm.at[pl.ds(g * CH, CH)], idx_v)
            pltpu.sync_copy(ones_v, acc_sh.at[idx_v], add=True)

        return _

    lax.fori_loop(0, (G_TOTAL // NC + NS - 1) // NS, body, None)
    plsc.subcore_barrier()

    pltpu.sync_copy(acc_sh.at[pl.ds(s * ROWS_P, ROWS_P)],
                    deg_hbm.at[c, pl.ds(s * ROWS_P, ROWS_P)])


@functools.partial(
    pl.kernel,
    out_type=jax.ShapeDtypeStruct((NC, N, HH), jnp.float32),
    mesh=_mesh,
    scratch_types=[
        pltpu.VMEM_SHARED((N, HH), jnp.float32),
        pltpu.VMEM((CH,), jnp.int32),
        pltpu.VMEM((CH,), jnp.int32),
        pltpu.VMEM((CH, HH), jnp.float32),
        pltpu.SemaphoreType.DMA,
    ],
)
def _agg_kernel(y_a, y_b, src_hbm, dst_hbm, out_hbm, acc_sh, sidx, didx, rows, sem):
    c = lax.axis_index("c")
    s = lax.axis_index("s")

    def _run(tab):
        def _cp(row0, nrows):
            pltpu.sync_copy(tab.at[pl.ds(row0, nrows)],
                            acc_sh.at[pl.ds(row0, nrows)])

        _rows_init(_cp, s)
        plsc.subcore_barrier()

        def body(k, _):
            g = s + k * NS

            @pl.when(g < G_TOTAL)
            def _():
                pltpu.sync_copy(src_hbm.at[pl.ds(g * CH, CH)], sidx)
                pltpu.sync_copy(dst_hbm.at[pl.ds(g * CH, CH)], didx)
                pltpu.async_copy(tab.at[sidx], rows, sem).wait()
                pltpu.sync_copy(rows, acc_sh.at[didx], add=True)

            return _

        lax.fori_loop(0, K_MAX, body, None)
        plsc.subcore_barrier()

        def _out(row0, nrows):
            pltpu.sync_copy(acc_sh.at[pl.ds(row0, nrows)],
                            out_hbm.at[c, pl.ds(row0, nrows)])

        _rows_init(_out, s)

    @pl.when(c == 0)
    def _():
        _run(y_a)

    @pl.when(c == 1)
    def _():
        _run(y_b)


@functools.partial(
    pl.kernel,
    out_type=jax.ShapeDtypeStruct((NP,), jnp.float32),
    mesh=_mesh,
    scratch_types=[
        pltpu.VMEM_SHARED((NP,), jnp.float32),
        pltpu.VMEM((CH,), jnp.int32),
        pltpu.VMEM((CH,), jnp.int32),
        pltpu.VMEM((CH,), jnp.float32),
        pltpu.VMEM((ROWS_P,), jnp.float32),
        pltpu.VMEM((ROWS_P,), jnp.float32),
        pltpu.VMEM((L,), jnp.float32),
        pltpu.SemaphoreType.DMA,
    ],
)
def _final_kernel(z_hbm, dinv_hbm, bo_hbm, src_hbm, dst_hbm, out_hbm,
                  acc_sh, sidx, didx, vals, acc_v, dinv_v, bo_v, sem):
    c = lax.axis_index("c")
    s = lax.axis_index("s")

    pltpu.sync_copy(z_hbm.at[pl.ds(s * ROWS_P, ROWS_P)],
                    acc_sh.at[pl.ds(s * ROWS_P, ROWS_P)])
    pltpu.sync_copy(bo_hbm, bo_v)
    plsc.subcore_barrier()

    def body(k, _):
        g = s + k * NS

        @pl.when(g < G_TOTAL)
        def _():
            pltpu.sync_copy(src_hbm.at[pl.ds(g * CH, CH)], sidx)
            pltpu.sync_copy(dst_hbm.at[pl.ds(g * CH, CH)], didx)
            pltpu.async_copy(z_hbm.at[sidx], vals, sem).wait()
            pltpu.sync_copy(vals, acc_sh.at[didx], add=True)

        return _

    lax.fori_loop(0, K_MAX, body, None)
    plsc.subcore_barrier()

    @pl.when(c == 0)
    def _():
        row0 = s * ROWS_P
        pltpu.sync_copy(acc_sh.at[pl.ds(row0, ROWS_P)], acc_v)
        pltpu.sync_copy(dinv_hbm.at[pl.ds(row0, ROWS_P)], dinv_v)
        bo = bo_v[...]
        for i in range(ROWS_P // L):
            a = acc_v[pl.ds(i * L, L)]
            d = dinv_v[pl.ds(i * L, L)]
            val = d * a + bo
            acc_v[pl.ds(i * L, L)] = 1.0 / (1.0 + jnp.exp(-val))
        pltpu.sync_copy(acc_v, out_hbm.at[pl.ds(row0, ROWS_P)])


R = 1000


def _s0_body(x_ref, Wp_ref, bp_ref, W0_ref, dinv_ref, h_ref, y_ref):
    h = jnp.dot(x_ref[...], Wp_ref[...],
                preferred_element_type=jnp.float32) + bp_ref[...]
    h_ref[...] = h
    y_ref[0] = dinv_ref[...] * jnp.dot(h, W0_ref[...],
                                       preferred_element_type=jnp.float32)


_stage0 = pl.pallas_call(
    _s0_body,
    grid=(N // R, 2),
    in_specs=[
        pl.BlockSpec((R, D_IN), lambda i, j: (i, 0)),
        pl.BlockSpec((D_IN, H), lambda i, j: (0, 0)),
        pl.BlockSpec((1, H), lambda i, j: (0, 0)),
        pl.BlockSpec((H, HH), lambda i, j: (0, j)),
        pl.BlockSpec((R, 1), lambda i, j: (i, 0)),
    ],
    out_specs=[
        pl.BlockSpec((R, H), lambda i, j: (i, 0)),
        pl.BlockSpec((1, R, HH), lambda i, j: (j, i, 0)),
    ],
    out_shape=[
        jax.ShapeDtypeStruct((N, H), jnp.float32),
        jax.ShapeDtypeStruct((2, N, HH), jnp.float32),
    ],
)


def _sk_body(agg_a, agg_b, hprev, dinv, b_ref, W_ref, hout, yout):
    aggf = jnp.concatenate([agg_a[0], agg_b[0]], axis=1)
    hn = jnp.maximum(dinv[...] * aggf + b_ref[...], 0.0) + hprev[...]
    hout[...] = hn
    yout[0] = dinv[...] * jnp.dot(hn, W_ref[...],
                                  preferred_element_type=jnp.float32)


_stage_k = pl.pallas_call(
    _sk_body,
    grid=(N // R, 2),
    in_specs=[
        pl.BlockSpec((1, R, HH), lambda i, j: (0, i, 0)),
        pl.BlockSpec((1, R, HH), lambda i, j: (1, i, 0)),
        pl.BlockSpec((R, H), lambda i, j: (i, 0)),
        pl.BlockSpec((R, 1), lambda i, j: (i, 0)),
        pl.BlockSpec((1, H), lambda i, j: (0, 0)),
        pl.BlockSpec((H, HH), lambda i, j: (0, j)),
    ],
    out_specs=[
        pl.BlockSpec((R, H), lambda i, j: (i, 0)),
        pl.BlockSpec((1, R, HH), lambda i, j: (j, i, 0)),
    ],
    out_shape=[
        jax.ShapeDtypeStruct((N, H), jnp.float32),
        jax.ShapeDtypeStruct((2, N, HH), jnp.float32),
    ],
)


def _s3_body(agg_a, agg_b, hprev, dinv, b_ref, Wo_ref, z_out):
    aggf = jnp.concatenate([agg_a[0], agg_b[0]], axis=1)
    h3 = jnp.maximum(dinv[...] * aggf + b_ref[...], 0.0) + hprev[...]
    z_out[...] = dinv[...] * jnp.dot(h3, Wo_ref[...],
                                     preferred_element_type=jnp.float32)


_stage3 = pl.pallas_call(
    _s3_body,
    grid=(N // R,),
    in_specs=[
        pl.BlockSpec((1, R, HH), lambda i: (0, i, 0)),
        pl.BlockSpec((1, R, HH), lambda i: (1, i, 0)),
        pl.BlockSpec((R, H), lambda i: (i, 0)),
        pl.BlockSpec((R, 1), lambda i: (i, 0)),
        pl.BlockSpec((1, H), lambda i: (0, 0)),
        pl.BlockSpec((H, 1), lambda i: (0, 0)),
    ],
    out_specs=pl.BlockSpec((R, 1), lambda i: (i, 0)),
    out_shape=jax.ShapeDtypeStruct((N, 1), jnp.float32),
)


def kernel(x, edge_index, batch, Wp, bp, W0, b0, W1, b1, W2, b2, Wo, bo):
    del batch
    src = edge_index[0].astype(jnp.int32)
    dst = edge_index[1].astype(jnp.int32)

    deg2 = _deg_kernel(dst)
    dinv_p = lax.rsqrt(deg2[0] + deg2[1])
    dinv = dinv_p[:N].reshape(N, 1)

    h, y = _stage0(x, Wp, bp.reshape(1, H), W0, dinv)
    agg = _agg_kernel(y[0], y[1], src, dst)
    h, y = _stage_k(agg, agg, h, dinv, b0.reshape(1, H), W1)
    agg = _agg_kernel(y[0], y[1], src, dst)
    h, y = _stage_k(agg, agg, h, dinv, b1.reshape(1, H), W2)
    agg = _agg_kernel(y[0], y[1], src, dst)
    z = _stage3(agg, agg, h, dinv, b2.reshape(1, H), Wo)

    bo16 = jnp.broadcast_to(bo, (L,)).astype(jnp.float32)
    z_p = jnp.concatenate([z.reshape(N), jnp.zeros((NP - N,), jnp.float32)])
    out = _final_kernel(z_p, dinv_p, bo16, src, dst)
    return out[:N]
